# Initial kernel scaffold; baseline (speedup 1.0000x reference)
import jax
import jax.numpy as jnp
from jax.experimental import pallas as pl


def kernel(x, w):
    raise NotImplementedError("write your pallas kernel here")


# speedup vs baseline: 1.0262x; 1.0262x over previous
import functools

import jax
import jax.numpy as jnp
from jax import lax
from jax.experimental import pallas as pl
from jax.experimental.pallas import tpu as pltpu
from jax.experimental.pallas import tpu_sc as plsc


def _row_sumsq(xb):
    x2 = xb * xb
    v = x2[:, :128] + x2[:, 128:]
    acc = v[:, 0:8]
    for k in range(1, 16):
        acc = acc + v[:, 8 * k:8 * k + 8]
    t = acc[:, :4] + acc[:, 4:]
    t = t[:, :2] + t[:, 2:]
    return t[:, 0:1] + t[:, 1:2]


def _argmin_body(x_ref, w_ref, out_ref, *, bn, half_tiles):
    xb = x_ref[...]
    bm = xb.shape[0]
    a = _row_sumsq(xb)

    def scan_half(h):
        bmin = jnp.full((bm, 1), jnp.inf, jnp.float32)
        bidx = jnp.zeros((bm, 1), jnp.int32)
        for t in range(half_tiles):
            n = h * half_tiles + t
            wt = w_ref[:, n * bn:(n + 1) * bn]
            m2 = jnp.dot(xb, wt, preferred_element_type=jnp.float32)
            w2 = jnp.sum(wt * wt, axis=0, keepdims=True)
            d = (a - 2.0 * m2) + w2
            tmin = jnp.min(d, axis=1, keepdims=True)
            iota = lax.broadcasted_iota(jnp.int32, d.shape, 1) + n * bn
            tidx = jnp.min(jnp.where(d == tmin, iota, jnp.int32(2**30)),
                           axis=1, keepdims=True)
            better = tmin < bmin
            bidx = jnp.where(better, tidx, bidx)
            bmin = jnp.where(better, tmin, bmin)
        return bmin, bidx

    m0, i0 = scan_half(0)
    m1, i1 = scan_half(1)
    u = lax.bitcast_convert_type(m0, jnp.int32)
    lo = u & 0xFFFF
    up = (lo > 0x8000) | ((lo == 0x8000) & (((u >> 16) & 1) == 0))
    u2 = (u & jnp.int32(-65536)) + jnp.where(up, jnp.int32(65536), jnp.int32(0))
    m0r = lax.bitcast_convert_type(u2, jnp.float32)
    take1 = m1 < m0r
    out_ref[0, 0, :] = jnp.where(take1, i1, i0)[:, 0]


def _argmin_indices(flat, w, bm=512, bn=512):
    m, k = flat.shape
    n = w.shape[1]
    grid = m // bm
    out = pl.pallas_call(
        functools.partial(_argmin_body, bn=bn, half_tiles=n // (2 * bn)),
        grid=(grid,),
        in_specs=[
            pl.BlockSpec((bm, k), lambda i: (i, 0)),
            pl.BlockSpec((k, n), lambda i: (0, 0)),
        ],
        out_specs=pl.BlockSpec((1, 1, bm), lambda i: (i, 0, 0)),
        out_shape=jax.ShapeDtypeStruct((grid, 1, bm), jnp.int32),
    )(flat, w)
    return out.reshape(-1)


def _make_sc_gather(v, d, b):
    info = plsc.get_sparse_core_info()
    nw = info.num_cores * info.num_subcores
    nc = info.num_cores
    b_per_w = b // nw
    mesh = plsc.VectorSubcoreMesh(core_axis_name="c", subcore_axis_name="s")

    @functools.partial(
        pl.kernel, mesh=mesh,
        out_type=jax.ShapeDtypeStruct((b, d), jnp.float32),
        scratch_types=[
            pltpu.VMEM((b_per_w,), jnp.int32),
            pltpu.VMEM((b_per_w, d), jnp.float32),
            pltpu.SemaphoreType.DMA,
        ],
    )
    def gather(table_hbm, idx_hbm, out_hbm, idx_v, rows_v, sem):
        wid = lax.axis_index("s") * nc + lax.axis_index("c")
        base = wid * b_per_w
        pltpu.sync_copy(idx_hbm.at[pl.ds(base, b_per_w)], idx_v)
        pltpu.async_copy(table_hbm.at[idx_v], rows_v, sem).wait()
        pltpu.sync_copy(rows_v, out_hbm.at[pl.ds(base, b_per_w)])

    return gather


def kernel(x, w):
    embedding_dim, num_codes = w.shape
    flat = x.reshape(-1, embedding_dim)
    idx = _argmin_indices(flat, w)
    wt = w.T
    gather = _make_sc_gather(num_codes, embedding_dim, flat.shape[0])
    quant = gather(wt, idx)
    return idx.reshape(x.shape[:-1]), quant.reshape(x.shape)


# speedup vs baseline: 1.0856x; 1.0579x over previous
import functools

import jax
import jax.numpy as jnp
from jax import lax
from jax.experimental import pallas as pl
from jax.experimental.pallas import tpu as pltpu
from jax.experimental.pallas import tpu_sc as plsc


def _row_sumsq(xb):
    x2 = xb * xb
    v = x2[:, :128] + x2[:, 128:]
    acc = v[:, 0:8]
    for k in range(1, 16):
        acc = acc + v[:, 8 * k:8 * k + 8]
    t = acc[:, :4] + acc[:, 4:]
    t = t[:, :2] + t[:, 2:]
    return t[:, 0:1] + t[:, 1:2]


def _argmin_body(x_ref, w_ref, out_ref, w2_ref, *, bn, half_tiles):
    i = pl.program_id(0)
    n_tiles = 2 * half_tiles

    @pl.when(i == 0)
    def _():
        for n in range(n_tiles):
            wt = w_ref[:, n * bn:(n + 1) * bn]
            w2_ref[:, n * bn:(n + 1) * bn] = jnp.sum(wt * wt, axis=0,
                                                     keepdims=True)

    xb = x_ref[...]
    bm = xb.shape[0]
    a = _row_sumsq(xb)
    xn2 = -2.0 * xb

    def scan_half(h):
        bmin = jnp.full((bm, 1), jnp.inf, jnp.float32)
        bidx = jnp.zeros((bm, 1), jnp.int32)
        for t in range(half_tiles):
            n = h * half_tiles + t
            wt = w_ref[:, n * bn:(n + 1) * bn]
            m2n = jnp.dot(xn2, wt, preferred_element_type=jnp.float32)
            d = (a + m2n) + w2_ref[:, n * bn:(n + 1) * bn]
            tmin = jnp.min(d, axis=1, keepdims=True)
            iota = lax.broadcasted_iota(jnp.int32, d.shape, 1)
            tloc = jnp.min(jnp.where(d == tmin, iota, jnp.int32(2**30)),
                           axis=1, keepdims=True)
            tidx = tloc + n * bn
            better = tmin < bmin
            bidx = jnp.where(better, tidx, bidx)
            bmin = jnp.where(better, tmin, bmin)
        return bmin, bidx

    m0, i0 = scan_half(0)
    m1, i1 = scan_half(1)
    u = lax.bitcast_convert_type(m0, jnp.int32)
    lo = u & 0xFFFF
    up = (lo > 0x8000) | ((lo == 0x8000) & (((u >> 16) & 1) == 0))
    u2 = (u & jnp.int32(-65536)) + jnp.where(up, jnp.int32(65536), jnp.int32(0))
    m0r = lax.bitcast_convert_type(u2, jnp.float32)
    take1 = m1 < m0r
    out_ref[0, 0, :] = jnp.where(take1, i1, i0)[:, 0]


def _argmin_indices(flat, w, bm=512, bn=512):
    m, k = flat.shape
    n = w.shape[1]
    grid = m // bm
    out = pl.pallas_call(
        functools.partial(_argmin_body, bn=bn, half_tiles=n // (2 * bn)),
        grid=(grid,),
        in_specs=[
            pl.BlockSpec((bm, k), lambda i: (i, 0)),
            pl.BlockSpec((k, n), lambda i: (0, 0)),
        ],
        out_specs=pl.BlockSpec((1, 1, bm), lambda i: (i, 0, 0)),
        out_shape=jax.ShapeDtypeStruct((grid, 1, bm), jnp.int32),
        scratch_shapes=[pltpu.VMEM((1, n), jnp.float32)],
    )(flat, w)
    return out.reshape(-1)


def _make_sc_gather(v, d, b):
    info = plsc.get_sparse_core_info()
    nw = info.num_cores * info.num_subcores
    nc = info.num_cores
    b_per_w = b // nw
    mesh = plsc.VectorSubcoreMesh(core_axis_name="c", subcore_axis_name="s")

    @functools.partial(
        pl.kernel, mesh=mesh,
        out_type=jax.ShapeDtypeStruct((b, d), jnp.float32),
        scratch_types=[
            pltpu.VMEM((b_per_w,), jnp.int32),
            pltpu.VMEM((b_per_w, d), jnp.float32),
            pltpu.SemaphoreType.DMA,
        ],
    )
    def gather(table_hbm, idx_hbm, out_hbm, idx_v, rows_v, sem):
        wid = lax.axis_index("s") * nc + lax.axis_index("c")
        base = wid * b_per_w
        pltpu.sync_copy(idx_hbm.at[pl.ds(base, b_per_w)], idx_v)
        pltpu.async_copy(table_hbm.at[idx_v], rows_v, sem).wait()
        pltpu.sync_copy(rows_v, out_hbm.at[pl.ds(base, b_per_w)])

    return gather


def kernel(x, w):
    embedding_dim, num_codes = w.shape
    flat = x.reshape(-1, embedding_dim)
    idx = _argmin_indices(flat, w)
    wt = w.T
    gather = _make_sc_gather(num_codes, embedding_dim, flat.shape[0])
    quant = gather(wt, idx)
    return idx.reshape(x.shape[:-1]), quant.reshape(x.shape)


# speedup vs baseline: 1.1574x; 1.0661x over previous
import functools

import jax
import jax.numpy as jnp
from jax import lax
from jax.experimental import pallas as pl
from jax.experimental.pallas import tpu as pltpu
from jax.experimental.pallas import tpu_sc as plsc


def _row_sumsq(xb):
    x2 = xb * xb
    v = x2[:, :128] + x2[:, 128:]
    acc = v[:, 0:8]
    for k in range(1, 16):
        acc = acc + v[:, 8 * k:8 * k + 8]
    t = acc[:, :4] + acc[:, 4:]
    t = t[:, :2] + t[:, 2:]
    return t[:, 0:1] + t[:, 1:2]


def _argmin_body(x_ref, w_ref, out_ref, w2_ref, *, bn, half_tiles):
    i = pl.program_id(0)
    n_tiles = 2 * half_tiles

    @pl.when(i == 0)
    def _():
        for n in range(n_tiles):
            wt = w_ref[:, n * bn:(n + 1) * bn]
            w2_ref[:, n * bn:(n + 1) * bn] = jnp.sum(wt * wt, axis=0,
                                                     keepdims=True)

    xb = x_ref[...]
    bm = xb.shape[0]
    a = _row_sumsq(xb)
    xn2 = -2.0 * xb

    def scan_half(h):
        bmin = jnp.full((bm, 1), jnp.inf, jnp.float32)
        bidx = jnp.zeros((bm, 1), jnp.int32)
        for t in range(half_tiles):
            n = h * half_tiles + t
            wt = w_ref[:, n * bn:(n + 1) * bn]
            m2n = jnp.dot(xn2, wt, preferred_element_type=jnp.float32)
            d = (a + m2n) + w2_ref[:, n * bn:(n + 1) * bn]
            tmin = jnp.min(d, axis=1, keepdims=True)
            iota = lax.broadcasted_iota(jnp.int32, d.shape, 1)
            tloc = jnp.min(jnp.where(d == tmin, iota, jnp.int32(2**30)),
                           axis=1, keepdims=True)
            tidx = tloc + n * bn
            better = tmin < bmin
            bidx = jnp.where(better, tidx, bidx)
            bmin = jnp.where(better, tmin, bmin)
        return bmin, bidx

    m0, i0 = scan_half(0)
    m1, i1 = scan_half(1)
    u = lax.bitcast_convert_type(m0, jnp.int32)
    lo = u & 0xFFFF
    up = (lo > 0x8000) | ((lo == 0x8000) & (((u >> 16) & 1) == 0))
    u2 = (u & jnp.int32(-65536)) + jnp.where(up, jnp.int32(65536), jnp.int32(0))
    m0r = lax.bitcast_convert_type(u2, jnp.float32)
    take1 = m1 < m0r
    out_ref[0, 0, :] = jnp.where(take1, i1, i0)[:, 0]


def _argmin_indices(flat, w, bm=512, bn=1024):
    m, k = flat.shape
    n = w.shape[1]
    grid = m // bm
    out = pl.pallas_call(
        functools.partial(_argmin_body, bn=bn, half_tiles=n // (2 * bn)),
        grid=(grid,),
        in_specs=[
            pl.BlockSpec((bm, k), lambda i: (i, 0)),
            pl.BlockSpec((k, n), lambda i: (0, 0)),
        ],
        out_specs=pl.BlockSpec((1, 1, bm), lambda i: (i, 0, 0)),
        out_shape=jax.ShapeDtypeStruct((grid, 1, bm), jnp.int32),
        scratch_shapes=[pltpu.VMEM((1, n), jnp.float32)],
    )(flat, w)
    return out.reshape(-1)


def _make_sc_gather(v, d, b):
    info = plsc.get_sparse_core_info()
    nw = info.num_cores * info.num_subcores
    nc = info.num_cores
    b_per_w = b // nw
    mesh = plsc.VectorSubcoreMesh(core_axis_name="c", subcore_axis_name="s")

    @functools.partial(
        pl.kernel, mesh=mesh,
        out_type=jax.ShapeDtypeStruct((b, d), jnp.float32),
        scratch_types=[
            pltpu.VMEM((b_per_w,), jnp.int32),
            pltpu.VMEM((b_per_w, d), jnp.float32),
            pltpu.SemaphoreType.DMA,
        ],
    )
    def gather(table_hbm, idx_hbm, out_hbm, idx_v, rows_v, sem):
        wid = lax.axis_index("s") * nc + lax.axis_index("c")
        base = wid * b_per_w
        pltpu.sync_copy(idx_hbm.at[pl.ds(base, b_per_w)], idx_v)
        pltpu.async_copy(table_hbm.at[idx_v], rows_v, sem).wait()
        pltpu.sync_copy(rows_v, out_hbm.at[pl.ds(base, b_per_w)])

    return gather


def kernel(x, w):
    embedding_dim, num_codes = w.shape
    flat = x.reshape(-1, embedding_dim)
    idx = _argmin_indices(flat, w)
    wt = w.T
    gather = _make_sc_gather(num_codes, embedding_dim, flat.shape[0])
    quant = gather(wt, idx)
    return idx.reshape(x.shape[:-1]), quant.reshape(x.shape)
